# baseline (speedup 1.0000x reference)
import functools

import jax
import jax.numpy as jnp
from jax.experimental import pallas as pl
from jax.experimental.pallas import tpu as pltpu


def _tile_kernel(f_ref, s_ref, o_ref):
    logits = jnp.dot(f_ref[...], s_ref[...].T, preferred_element_type=jnp.float32)
    o_ref[...] = jax.nn.sigmoid(logits)


def _run(fan_factors, shopkeeper_factors):
    m, d = fan_factors.shape
    n = shopkeeper_factors.shape[0]
    bm = 1024
    grid = (m // bm,)
    return pl.pallas_call(
        _tile_kernel,
        grid=grid,
        in_specs=[
            pl.BlockSpec((bm, d), lambda i: (i, 0)),
            pl.BlockSpec((n, d), lambda i: (0, 0)),
        ],
        out_specs=pl.BlockSpec((bm, n), lambda i: (i, 0)),
        out_shape=jax.ShapeDtypeStruct((m, n), jnp.float32),
        compiler_params=pltpu.CompilerParams(
            dimension_semantics=("parallel",),
        ),
    )(fan_factors, shopkeeper_factors)


def kernel(n_fans, n_shopkeepers, fan_factors, shopkeeper_factors):
    return _run(fan_factors, shopkeeper_factors)


# speedup vs baseline: 1.0985x; 1.0985x over previous
import functools

import jax
import jax.numpy as jnp
from jax.experimental import pallas as pl
from jax.experimental.pallas import tpu as pltpu


def _tile_kernel(f_ref, s_ref, o_ref):
    logits = jnp.dot(f_ref[...], s_ref[...].T, preferred_element_type=jnp.float32)
    o_ref[...] = 0.5 * jnp.tanh(0.5 * logits) + 0.5


def _run(fan_factors, shopkeeper_factors):
    m, d = fan_factors.shape
    n = shopkeeper_factors.shape[0]
    bm = 1024
    grid = (m // bm,)
    return pl.pallas_call(
        _tile_kernel,
        grid=grid,
        in_specs=[
            pl.BlockSpec((bm, d), lambda i: (i, 0)),
            pl.BlockSpec((n, d), lambda i: (0, 0)),
        ],
        out_specs=pl.BlockSpec((bm, n), lambda i: (i, 0)),
        out_shape=jax.ShapeDtypeStruct((m, n), jnp.float32),
        compiler_params=pltpu.CompilerParams(
            dimension_semantics=("parallel",),
        ),
    )(fan_factors, shopkeeper_factors)


def kernel(n_fans, n_shopkeepers, fan_factors, shopkeeper_factors):
    return _run(fan_factors, shopkeeper_factors)


# speedup vs baseline: 1.1003x; 1.0017x over previous
import functools

import jax
import jax.numpy as jnp
from jax.experimental import pallas as pl
from jax.experimental.pallas import tpu as pltpu


def _tile_kernel(f_ref, s_ref, o_ref):
    logits = jnp.dot(f_ref[...], s_ref[...].T, preferred_element_type=jnp.float32)
    o_ref[...] = 0.5 * jnp.tanh(0.5 * logits) + 0.5


def _run(fan_factors, shopkeeper_factors):
    m, d = fan_factors.shape
    n = shopkeeper_factors.shape[0]
    bm = 512
    grid = (m // bm,)
    return pl.pallas_call(
        _tile_kernel,
        grid=grid,
        in_specs=[
            pl.BlockSpec((bm, d), lambda i: (i, 0)),
            pl.BlockSpec((n, d), lambda i: (0, 0)),
        ],
        out_specs=pl.BlockSpec((bm, n), lambda i: (i, 0)),
        out_shape=jax.ShapeDtypeStruct((m, n), jnp.float32),
        compiler_params=pltpu.CompilerParams(
            dimension_semantics=("parallel",),
        ),
    )(fan_factors, shopkeeper_factors)


def kernel(n_fans, n_shopkeepers, fan_factors, shopkeeper_factors):
    return _run(fan_factors, shopkeeper_factors)


# speedup vs baseline: 1.1004x; 1.0001x over previous
import jax
import jax.numpy as jnp
from jax.experimental import pallas as pl
from jax.experimental.pallas import tpu as pltpu

_NBUF = 4


def _tile_kernel(f_ref, s_ref, o_hbm, scratch, sems):
    i = pl.program_id(0)
    nsteps = pl.num_programs(0)
    bm = f_ref.shape[0]
    slot = jax.lax.rem(i, _NBUF)

    @pl.when(i >= _NBUF)
    def _wait_slot():
        pltpu.make_async_copy(
            scratch.at[slot], o_hbm.at[pl.ds((i - _NBUF) * bm, bm), :], sems.at[slot]
        ).wait()

    logits = jnp.dot(f_ref[...], s_ref[...].T, preferred_element_type=jnp.float32)
    scratch[slot] = 0.5 * jnp.tanh(0.5 * logits) + 0.5
    pltpu.make_async_copy(
        scratch.at[slot], o_hbm.at[pl.ds(i * bm, bm), :], sems.at[slot]
    ).start()

    @pl.when(i == nsteps - 1)
    def _drain():
        for j in range(_NBUF):
            k = nsteps - _NBUF + j
            pltpu.make_async_copy(
                scratch.at[jax.lax.rem(k, _NBUF)],
                o_hbm.at[pl.ds(k * bm, bm), :],
                sems.at[jax.lax.rem(k, _NBUF)],
            ).wait()


def _run(fan_factors, shopkeeper_factors):
    m, d = fan_factors.shape
    n = shopkeeper_factors.shape[0]
    bm = 512
    grid = (m // bm,)
    return pl.pallas_call(
        _tile_kernel,
        grid=grid,
        in_specs=[
            pl.BlockSpec((bm, d), lambda i: (i, 0)),
            pl.BlockSpec((n, d), lambda i: (0, 0)),
        ],
        out_specs=pl.BlockSpec(memory_space=pltpu.MemorySpace.HBM),
        out_shape=jax.ShapeDtypeStruct((m, n), jnp.float32),
        scratch_shapes=[
            pltpu.VMEM((_NBUF, bm, n), jnp.float32),
            pltpu.SemaphoreType.DMA((_NBUF,)),
        ],
    )(fan_factors, shopkeeper_factors)


def kernel(n_fans, n_shopkeepers, fan_factors, shopkeeper_factors):
    return _run(fan_factors, shopkeeper_factors)
